# baseline (speedup 1.0000x reference)
import functools

import jax
import jax.numpy as jnp
from jax import lax
from jax.experimental import pallas as pl
from jax.experimental.pallas import tpu as pltpu
from jax.experimental.pallas import tpu_sc as plsc

NC = 2
NS = 16
L = 16

NCG = 8
K = 16
KW = K // 2
NNH = 2
NEH = 2
CE = 2000


def _sc_body(n, e, rh, akp, row_hbm, col_hbm, xt_hbm, mv_hbm, out_hbm,
             rowv, colv, xrep, aggr, mbuf):
    wid = lax.axis_index("s") * NC + lax.axis_index("c")
    cg = wid % NCG
    nh = (wid // NCG) % NNH
    eh = wid // (NCG * NNH)
    nlo = nh * rh
    eshare = e // NEH
    ebase = eh * eshare
    nchunks = eshare // CE
    nw = n * KW

    pltpu.sync_copy(xt_hbm.at[pl.ds(cg * nw, nw)], xrep.at[pl.ds(0, nw)])
    xrep[pl.ds(nw, L)] = jnp.zeros((L,), dtype=jnp.int32)
    pltpu.sync_copy(mv_hbm, mbuf)
    mv = mbuf[...]

    def init(j, _):
        aggr[pl.ds(j * L, L)] = jnp.zeros((L,), dtype=jnp.int32)
        return 0
    lax.fori_loop(0, akp // L, init, 0)

    def chunk_body(ch, _):
        base = ebase + ch * CE
        pltpu.sync_copy(row_hbm.at[pl.ds(base, CE)], rowv)
        pltpu.sync_copy(col_hbm.at[pl.ds(base, CE)], colv)

        def acc(j, _):
            rv = rowv[pl.ds(j * L, L)]
            rloc = jnp.where((rv >= nlo) & (rv < nlo + rh), rv - nlo, rh)
            aoff = rloc * KW
            xoff = colv[pl.ds(j * L, L)] * KW
            for i in range(L):
                ao = aoff[i]

                @pl.when(ao < rh * KW)
                def _():
                    xo = xoff[i]
                    a = aggr[pl.ds(ao, L)]
                    v = xrep[pl.ds(xo, L)] & mv
                    ml = jnp.maximum(a & 0xFFFF, v & 0xFFFF)
                    mh = jnp.maximum(lax.shift_right_logical(a, 16),
                                     lax.shift_right_logical(v, 16))
                    aggr[pl.ds(ao, L)] = ml | lax.shift_left(mh, 16)
            return 0
        lax.fori_loop(0, CE // L, acc, 0)
        return 0

    lax.fori_loop(0, nchunks, chunk_body, 0)

    pltpu.sync_copy(aggr,
                    out_hbm.at[pl.ds(((eh * NCG + cg) * NNH + nh) * akp,
                                     akp)])


def _dense_body(x_ref, k0_ref, k1_ref, w_ref, b_ref, o_ref):
    k = jnp.maximum(k0_ref[...], k1_ref[...])
    u = jnp.where(k >= 32768, k ^ 32768, k ^ 65535)
    val = lax.bitcast_convert_type(lax.shift_left(u, 16), jnp.float32)
    a = jnp.where(k == 0, 0.0, val) - x_ref[...]
    h = lax.dot_general(a, w_ref[...], (((1,), (1,)), ((), ())),
                        preferred_element_type=jnp.float32)
    h = h + b_ref[...]
    o_ref[...] = h * 0.5 * (1.0 + lax.erf(h * 0.7071067811865476))


def kernel(x, edge_index, W, b):
    Bv, N, C = x.shape
    C_out = W.shape[0]
    E = edge_index.shape[1]
    x_flat = x.reshape(N, C)
    row = edge_index[0]
    col = edge_index[1]
    rh = N // NNH

    AKP = (rh * KW + 2 * L + 7) // 8 * 8

    u = lax.bitcast_convert_type(x_flat.astype(jnp.bfloat16),
                                 jnp.uint16).astype(jnp.int32)
    key = jnp.where(u >= 32768, u ^ 65535, u ^ 32768)
    kp = key.reshape(N, NCG, KW, 2)
    xt = (kp[..., 0] | lax.shift_left(kp[..., 1], 16))
    xt = xt.transpose(1, 0, 2).reshape(NCG * N * KW)
    mvec = jnp.asarray([-1] * KW + [0] * (L - KW), dtype=jnp.int32)

    mesh = plsc.VectorSubcoreMesh(core_axis_name="c", subcore_axis_name="s",
                                  num_cores=NC, num_subcores=NS)
    sc = pl.kernel(
        functools.partial(_sc_body, N, E, rh, AKP),
        out_type=jax.ShapeDtypeStruct((NEH * NCG * NNH * AKP,), jnp.int32),
        mesh=mesh,
        scratch_types=[
            pltpu.VMEM((CE,), jnp.int32),
            pltpu.VMEM((CE,), jnp.int32),
            pltpu.VMEM((N * KW + L,), jnp.int32),
            pltpu.VMEM((AKP,), jnp.int32),
            pltpu.VMEM((L,), jnp.int32),
        ],
    )
    part = sc(row, col, xt, mvec)

    part = part.reshape(NEH, NCG, NNH, AKP)[:, :, :, :rh * KW]
    part = part.reshape(NEH, NCG, NNH, rh, KW)
    lo = part & 0xFFFF
    hi = lax.shift_right_logical(part, 16)
    keys = jnp.stack([lo, hi], axis=-1).reshape(NEH, NCG, NNH, rh, K)
    keys = keys.transpose(0, 2, 3, 1, 4).reshape(NEH, N, C)

    RB = 1000
    out = pl.pallas_call(
        _dense_body,
        grid=(N // RB,),
        in_specs=[
            pl.BlockSpec((RB, C), lambda i: (i, 0)),
            pl.BlockSpec((RB, C), lambda i: (i, 0)),
            pl.BlockSpec((RB, C), lambda i: (i, 0)),
            pl.BlockSpec((C_out, C), lambda i: (0, 0)),
            pl.BlockSpec((1, C_out), lambda i: (0, 0)),
        ],
        out_specs=pl.BlockSpec((RB, C_out), lambda i: (i, 0)),
        out_shape=jax.ShapeDtypeStruct((N, C_out), jnp.float32),
    )(x_flat, keys[0], keys[1], W, b.reshape(1, C_out))

    return out.reshape(Bv, N, C_out)


# speedup vs baseline: 1.1232x; 1.1232x over previous
import functools

import jax
import jax.numpy as jnp
from jax import lax
from jax.experimental import pallas as pl
from jax.experimental.pallas import tpu as pltpu
from jax.experimental.pallas import tpu_sc as plsc

NC = 2
NS = 16
L = 16

NCG = 8
K = 16
KW = K // 2
NNH = 2
NEH = 2
CE = 2000


def _sc_body(n, e, rh, akp, row_hbm, col_hbm, xt_hbm, mv_hbm, out_hbm,
             rowa, cola, rowb, colb, xrep, aggr, mbuf,
             sra, sca, srb, scb):
    wid = lax.axis_index("s") * NC + lax.axis_index("c")
    cg = wid % NCG
    nh = (wid // NCG) % NNH
    eh = wid // (NCG * NNH)
    nlo = nh * rh
    eshare = e // NEH
    ebase = eh * eshare
    nchunks = eshare // CE
    nw = n * KW

    pltpu.sync_copy(xt_hbm.at[pl.ds(cg * nw, nw)], xrep.at[pl.ds(0, nw)])
    xrep[pl.ds(nw, L)] = jnp.zeros((L,), dtype=jnp.int32)
    pltpu.sync_copy(mv_hbm, mbuf)
    mv = mbuf[...]

    def init(j, _):
        aggr[pl.ds(j * L, L)] = jnp.zeros((L,), dtype=jnp.int32)
        return 0
    lax.fori_loop(0, akp // L, init, 0)

    def start(ch, rbuf, cbuf, sr, sc_):
        base = ebase + ch * CE
        pltpu.make_async_copy(row_hbm.at[pl.ds(base, CE)], rbuf, sr).start()
        pltpu.make_async_copy(col_hbm.at[pl.ds(base, CE)], cbuf, sc_).start()

    def wait(rbuf, cbuf, sr, sc_):
        pltpu.make_async_copy(row_hbm.at[pl.ds(0, CE)], rbuf, sr).wait()
        pltpu.make_async_copy(col_hbm.at[pl.ds(0, CE)], cbuf, sc_).wait()

    def process(rowv, colv):
        def acc(j, _):
            rv = rowv[pl.ds(j * L, L)]
            rloc = jnp.where((rv >= nlo) & (rv < nlo + rh), rv - nlo, rh)
            aoff = rloc * KW
            xoff = colv[pl.ds(j * L, L)] * KW
            for i in range(L):
                ao = aoff[i]
                xo = xoff[i]
                a = aggr[pl.ds(ao, L)]
                v = xrep[pl.ds(xo, L)] & mv
                ml = jnp.maximum(a & 0xFFFF, v & 0xFFFF)
                mh = jnp.maximum(lax.shift_right_logical(a, 16),
                                 lax.shift_right_logical(v, 16))
                aggr[pl.ds(ao, L)] = ml | lax.shift_left(mh, 16)
            return 0
        lax.fori_loop(0, CE // L, acc, 0)

    npairs = nchunks // 2
    start(0, rowa, cola, sra, sca)

    def pair_body(g, _):
        start(2 * g + 1, rowb, colb, srb, scb)
        wait(rowa, cola, sra, sca)
        process(rowa, cola)

        @pl.when(g < npairs - 1)
        def _():
            start(2 * g + 2, rowa, cola, sra, sca)
        wait(rowb, colb, srb, scb)
        process(rowb, colb)
        return 0

    lax.fori_loop(0, npairs, pair_body, 0)

    pltpu.sync_copy(aggr,
                    out_hbm.at[pl.ds(((eh * NCG + cg) * NNH + nh) * akp,
                                     akp)])


def _dense_body(x_ref, k0_ref, k1_ref, w_ref, b_ref, o_ref):
    k = jnp.maximum(k0_ref[...], k1_ref[...])
    u = jnp.where(k >= 32768, k ^ 32768, k ^ 65535)
    val = lax.bitcast_convert_type(lax.shift_left(u, 16), jnp.float32)
    a = jnp.where(k == 0, 0.0, val) - x_ref[...]
    h = lax.dot_general(a, w_ref[...], (((1,), (1,)), ((), ())),
                        preferred_element_type=jnp.float32)
    h = h + b_ref[...]
    o_ref[...] = h * 0.5 * (1.0 + lax.erf(h * 0.7071067811865476))


def kernel(x, edge_index, W, b):
    Bv, N, C = x.shape
    C_out = W.shape[0]
    E = edge_index.shape[1]
    x_flat = x.reshape(N, C)
    row = edge_index[0]
    col = edge_index[1]
    rh = N // NNH

    AKP = (rh * KW + 2 * L + 7) // 8 * 8

    u = lax.bitcast_convert_type(x_flat.astype(jnp.bfloat16),
                                 jnp.uint16).astype(jnp.int32)
    key = jnp.where(u >= 32768, u ^ 65535, u ^ 32768)
    kp = key.reshape(N, NCG, KW, 2)
    xt = (kp[..., 0] | lax.shift_left(kp[..., 1], 16))
    xt = xt.transpose(1, 0, 2).reshape(NCG * N * KW)
    mvec = jnp.asarray([-1] * KW + [0] * (L - KW), dtype=jnp.int32)

    mesh = plsc.VectorSubcoreMesh(core_axis_name="c", subcore_axis_name="s",
                                  num_cores=NC, num_subcores=NS)
    sc = pl.kernel(
        functools.partial(_sc_body, N, E, rh, AKP),
        out_type=jax.ShapeDtypeStruct((NEH * NCG * NNH * AKP,), jnp.int32),
        mesh=mesh,
        scratch_types=[
            pltpu.VMEM((CE,), jnp.int32),
            pltpu.VMEM((CE,), jnp.int32),
            pltpu.VMEM((CE,), jnp.int32),
            pltpu.VMEM((CE,), jnp.int32),
            pltpu.VMEM((N * KW + L,), jnp.int32),
            pltpu.VMEM((AKP,), jnp.int32),
            pltpu.VMEM((L,), jnp.int32),
            pltpu.SemaphoreType.DMA,
            pltpu.SemaphoreType.DMA,
            pltpu.SemaphoreType.DMA,
            pltpu.SemaphoreType.DMA,
        ],
    )
    part = sc(row, col, xt, mvec)

    part = part.reshape(NEH, NCG, NNH, AKP)[:, :, :, :rh * KW]
    part = part.reshape(NEH, NCG, NNH, rh, KW)
    lo = part & 0xFFFF
    hi = lax.shift_right_logical(part, 16)
    keys = jnp.stack([lo, hi], axis=-1).reshape(NEH, NCG, NNH, rh, K)
    keys = keys.transpose(0, 2, 3, 1, 4).reshape(NEH, N, C)

    RB = 1000
    out = pl.pallas_call(
        _dense_body,
        grid=(N // RB,),
        in_specs=[
            pl.BlockSpec((RB, C), lambda i: (i, 0)),
            pl.BlockSpec((RB, C), lambda i: (i, 0)),
            pl.BlockSpec((RB, C), lambda i: (i, 0)),
            pl.BlockSpec((C_out, C), lambda i: (0, 0)),
            pl.BlockSpec((1, C_out), lambda i: (0, 0)),
        ],
        out_specs=pl.BlockSpec((RB, C_out), lambda i: (i, 0)),
        out_shape=jax.ShapeDtypeStruct((N, C_out), jnp.float32),
    )(x_flat, keys[0], keys[1], W, b.reshape(1, C_out))

    return out.reshape(Bv, N, C_out)


# speedup vs baseline: 1.2763x; 1.1363x over previous
import functools

import jax
import jax.numpy as jnp
from jax import lax
from jax.experimental import pallas as pl
from jax.experimental.pallas import tpu as pltpu
from jax.experimental.pallas import tpu_sc as plsc

NC = 2
NS = 16
L = 16

NCG = 8
K = 16
KW = K // 2
NNH = 2
NEH = 2
CE = 2000


def _sc_body(n, e, rh, akp, row_hbm, col_hbm, xt_hbm, mv_hbm, out_hbm,
             rowa, cola, rowb, colb, xrep, aggr, mbuf,
             sra, sca, srb, scb):
    wid = lax.axis_index("s") * NC + lax.axis_index("c")
    cg = wid % NCG
    nh = (wid // NCG) % NNH
    eh = wid // (NCG * NNH)
    nlo = nh * rh
    eshare = e // NEH
    ebase = eh * eshare
    nchunks = eshare // CE
    nw = n * KW

    pltpu.sync_copy(xt_hbm.at[pl.ds(cg * nw, nw)], xrep.at[pl.ds(0, nw)])
    xrep[pl.ds(nw, L)] = jnp.zeros((L,), dtype=jnp.int32)
    pltpu.sync_copy(mv_hbm, mbuf)
    mv = mbuf[...]

    def init(j, _):
        aggr[pl.ds(j * L, L)] = jnp.zeros((L,), dtype=jnp.int32)
        return 0
    lax.fori_loop(0, akp // L + 2, init, 0)

    def start(ch, rbuf, cbuf, sr, sc_):
        base = ebase + ch * CE
        pltpu.make_async_copy(row_hbm.at[pl.ds(base, CE)], rbuf, sr).start()
        pltpu.make_async_copy(col_hbm.at[pl.ds(base, CE)], cbuf, sc_).start()

    def wait(rbuf, cbuf, sr, sc_):
        pltpu.make_async_copy(row_hbm.at[pl.ds(0, CE)], rbuf, sr).wait()
        pltpu.make_async_copy(col_hbm.at[pl.ds(0, CE)], cbuf, sc_).wait()

    def process(rowv, colv):
        def acc(j, _):
            rv = rowv[pl.ds(j * L, L)]
            rloc = jnp.where((rv >= nlo) & (rv < nlo + rh), rv - nlo, rh)
            aoff = rloc * KW
            xoff = colv[pl.ds(j * L, L)] * KW
            for i in range(L):
                ao = aoff[i]
                xo = xoff[i]
                a = aggr[pl.ds(ao, L)]
                v = xrep[pl.ds(xo, L)] & mv
                ml = jnp.maximum(a & 0xFFFF, v & 0xFFFF)
                mh = jnp.maximum(lax.shift_right_logical(a, 16),
                                 lax.shift_right_logical(v, 16))
                aggr[pl.ds(ao, L)] = ml | lax.shift_left(mh, 16)
            return 0
        lax.fori_loop(0, CE // L, acc, 0)

    npairs = nchunks // 2
    start(0, rowa, cola, sra, sca)

    def pair_body(g, _):
        start(2 * g + 1, rowb, colb, srb, scb)
        wait(rowa, cola, sra, sca)
        process(rowa, cola)

        @pl.when(g < npairs - 1)
        def _():
            start(2 * g + 2, rowa, cola, sra, sca)
        wait(rowb, colb, srb, scb)
        process(rowb, colb)
        return 0

    lax.fori_loop(0, npairs, pair_body, 0)

    pltpu.sync_copy(aggr.at[pl.ds(0, akp)],
                    out_hbm.at[pl.ds(((eh * NCG + cg) * NNH + nh) * akp,
                                     akp)])


def _dense_body(x_ref, p0_ref, p1_ref, w_ref, wp_ref, b_ref, o_ref):
    xw = lax.dot_general(x_ref[...], w_ref[...], (((1,), (1,)), ((), ())),
                         preferred_element_type=jnp.float32)
    acc = b_ref[...] - xw
    wp = wp_ref[...]
    p0 = p0_ref[...]
    p1 = p1_ref[...]
    for cg in range(NCG):
        k = jnp.maximum(p0[cg, 0], p1[cg, 0])
        kk = jnp.concatenate([k & 0xFFFF, lax.shift_right_logical(k, 16)],
                             axis=1)
        u = jnp.where(kk >= 32768, kk ^ 32768, kk ^ 65535)
        val = lax.bitcast_convert_type(lax.shift_left(u, 16), jnp.float32)
        a = jnp.where(kk == 0, 0.0, val)
        acc = acc + lax.dot_general(
            a, wp[cg], (((1,), (0,)), ((), ())),
            preferred_element_type=jnp.float32)
    o_ref[...] = acc * 0.5 * (1.0 + lax.erf(acc * 0.7071067811865476))


def kernel(x, edge_index, W, b):
    Bv, N, C = x.shape
    C_out = W.shape[0]
    E = edge_index.shape[1]
    x_flat = x.reshape(N, C)
    row = edge_index[0]
    col = edge_index[1]
    rh = N // NNH

    AKP = rh * KW

    u = lax.bitcast_convert_type(x_flat.astype(jnp.bfloat16),
                                 jnp.uint16).astype(jnp.int32)
    key = jnp.where(u >= 32768, u ^ 65535, u ^ 32768)
    kp = key.reshape(N, NCG, KW, 2)
    xt = (kp[..., 0] | lax.shift_left(kp[..., 1], 16))
    xt = xt.transpose(1, 0, 2).reshape(NCG * N * KW)
    mvec = jnp.asarray([-1] * KW + [0] * (L - KW), dtype=jnp.int32)

    mesh = plsc.VectorSubcoreMesh(core_axis_name="c", subcore_axis_name="s",
                                  num_cores=NC, num_subcores=NS)
    sc = pl.kernel(
        functools.partial(_sc_body, N, E, rh, AKP),
        out_type=jax.ShapeDtypeStruct((NEH * NCG * NNH * AKP,), jnp.int32),
        mesh=mesh,
        scratch_types=[
            pltpu.VMEM((CE,), jnp.int32),
            pltpu.VMEM((CE,), jnp.int32),
            pltpu.VMEM((CE,), jnp.int32),
            pltpu.VMEM((CE,), jnp.int32),
            pltpu.VMEM((N * KW + L,), jnp.int32),
            pltpu.VMEM((AKP + 2 * L,), jnp.int32),
            pltpu.VMEM((L,), jnp.int32),
            pltpu.SemaphoreType.DMA,
            pltpu.SemaphoreType.DMA,
            pltpu.SemaphoreType.DMA,
            pltpu.SemaphoreType.DMA,
        ],
    )
    part = sc(row, col, xt, mvec)

    part = part.reshape(NEH, NCG, NNH, rh, KW)

    perm = [cg * K + (2 * j if j < KW else 2 * (j - KW) + 1)
            for cg in range(NCG) for j in range(K)]
    wp = (W[:, jnp.asarray(perm, dtype=jnp.int32)]
          .reshape(C_out, NCG, K).transpose(1, 2, 0))

    RB = 1000
    nrb = rh // RB
    out = pl.pallas_call(
        _dense_body,
        grid=(N // RB,),
        in_specs=[
            pl.BlockSpec((RB, C), lambda i: (i, 0)),
            pl.BlockSpec((NCG, 1, RB, KW),
                         lambda i: (0, i // nrb, i % nrb, 0)),
            pl.BlockSpec((NCG, 1, RB, KW),
                         lambda i: (0, i // nrb, i % nrb, 0)),
            pl.BlockSpec((C_out, C), lambda i: (0, 0)),
            pl.BlockSpec((NCG, K, C_out), lambda i: (0, 0, 0)),
            pl.BlockSpec((1, C_out), lambda i: (0, 0)),
        ],
        out_specs=pl.BlockSpec((RB, C_out), lambda i: (i, 0)),
        out_shape=jax.ShapeDtypeStruct((N, C_out), jnp.float32),
    )(x_flat, part[0], part[1], W, wp, b.reshape(1, C_out))

    return out.reshape(Bv, N, C_out)
